# Initial kernel scaffold; baseline (speedup 1.0000x reference)
import jax
import jax.numpy as jnp
from jax.experimental import pallas as pl


def kernel(query, key, value, Wq, bq, Wv, bv, Wout, bout, hash_w):
    raise NotImplementedError("write your pallas kernel here")


# speedup vs baseline: 8.3049x; 8.3049x over previous
import functools

import jax
import jax.numpy as jnp
from jax import lax
from jax.experimental import pallas as pl
from jax.experimental.pallas import tpu as pltpu
from jax.experimental.pallas import tpu_sc as plsc

T, B, E, H = 4096, 2, 1024, 16
Dh = E // H
R, NH, C = 2, 16, 128
nC = T // C
BH = B * H
RBH = R * BH
SCALING = Dh ** -0.5
LOG2T = 12


TP = 1024
def _proj_body(xq_ref, xv_ref, wq_ref, wv_ref, bq_ref, bv_ref, q_ref, v_ref):
    xq = xq_ref[0]
    xv = xv_ref[0]
    wq = wq_ref[...]
    wv = wv_ref[...]
    dn = (((1,), (1,)), ((), ()))
    q = lax.dot_general(xq, wq, dn, preferred_element_type=jnp.float32)
    v = lax.dot_general(xv, wv, dn, preferred_element_type=jnp.float32)
    q_ref[0] = q + bq_ref[0, 0][None, :]
    v_ref[0] = v + bv_ref[0, 0][None, :]


def _proj(query, value, Wq, bq, Wv, bv):
    bq3 = bq.reshape(H, 1, Dh)
    bv3 = bv.reshape(H, 1, Dh)
    return pl.pallas_call(
        _proj_body,
        grid=(B, T // TP, H),
        in_specs=[
            pl.BlockSpec((1, TP, E), lambda b, t, h: (b, t, 0)),
            pl.BlockSpec((1, TP, E), lambda b, t, h: (b, t, 0)),
            pl.BlockSpec((Dh, E), lambda b, t, h: (h, 0)),
            pl.BlockSpec((Dh, E), lambda b, t, h: (h, 0)),
            pl.BlockSpec((1, 1, Dh), lambda b, t, h: (h, 0, 0)),
            pl.BlockSpec((1, 1, Dh), lambda b, t, h: (h, 0, 0)),
        ],
        out_specs=[
            pl.BlockSpec((1, TP, Dh), lambda b, t, h: (b * H + h, t, 0)),
            pl.BlockSpec((1, TP, Dh), lambda b, t, h: (b * H + h, t, 0)),
        ],
        out_shape=[
            jax.ShapeDtypeStruct((BH, T, Dh), jnp.float32),
            jax.ShapeDtypeStruct((BH, T, Dh), jnp.float32),
        ],
        compiler_params=pltpu.CompilerParams(
            dimension_semantics=("arbitrary", "arbitrary", "arbitrary")),
    )(query, value, Wq, Wv, bq3, bv3)


def _hashperm_body(q_ref, w_ref, undo_ref, meta_ref, dstg_ref):
    r = pl.program_id(0)
    bh = pl.program_id(1)
    q3 = q_ref[0].reshape(nC, C, Dh)
    w = w_ref[0, 0]
    wf = jnp.concatenate([w, -w], axis=1)
    rot = lax.dot_general(q3, wf, (((2,), (0,)), ((), ())),
                          preferred_element_type=jnp.float32)
    mx = jnp.max(rot, axis=-1, keepdims=True)
    lane = lax.broadcasted_iota(jnp.int32, (nC, C, NH), 2)
    codes = jnp.min(jnp.where(rot >= mx, lane, NH), axis=-1)

    iota_c = lax.broadcasted_iota(jnp.int32, (nC, NH, C), 1)
    oh_c = (codes[:, None, :] == iota_c).astype(jnp.float32)
    cnts = jnp.sum(oh_c, axis=2)
    hp = jax.lax.Precision.HIGHEST
    ii = lax.broadcasted_iota(jnp.int32, (nC, nC), 0)
    ik = lax.broadcasted_iota(jnp.int32, (nC, nC), 1)
    Lb = (ik < ii).astype(jnp.float32)
    S = lax.dot_general(Lb, cnts, (((1,), (0,)), ((), ())),
                        preferred_element_type=jnp.float32, precision=hp)
    totals = jnp.sum(cnts, axis=0, keepdims=True)
    cj = lax.broadcasted_iota(jnp.int32, (NH, NH), 0)
    ck = lax.broadcasted_iota(jnp.int32, (NH, NH), 1)
    Ub = (cj < ck).astype(jnp.float32)
    offs = lax.dot_general(totals, Ub, (((1,), (0,)), ((), ())),
                           preferred_element_type=jnp.float32, precision=hp)

    jlt = (lax.broadcasted_iota(jnp.int32, (nC, C, C), 2)
           < lax.broadcasted_iota(jnp.int32, (nC, C, C), 1))
    same = (codes[:, :, None] == codes[:, None, :]) & jlt
    wpick = jnp.sum(same.astype(jnp.float32), axis=-1)
    iota_j = lax.broadcasted_iota(jnp.int32, (nC, C, NH), 2)
    oh_j = (codes[:, :, None] == iota_j).astype(jnp.float32)
    base = jnp.sum(oh_j * (S + offs)[:, None, :], axis=-1)
    undo = (wpick + base + 0.5).astype(jnp.int32)

    ti = lax.broadcasted_iota(jnp.int32, (nC, C), 0) * C \
        + lax.broadcasted_iota(jnp.int32, (nC, C), 1)
    meta = codes * T + ti
    undo_ref[0] = undo
    meta_ref[0] = meta
    dstg_ref[0] = undo + (r * BH + bh) * T


def _hashperm(qbh, hash_w):
    return pl.pallas_call(
        _hashperm_body,
        grid=(R, BH),
        in_specs=[
            pl.BlockSpec((1, T, Dh), lambda r, bh: (bh, 0, 0)),
            pl.BlockSpec((1, 1, Dh, NH // 2), lambda r, bh: (r, bh % H, 0, 0)),
        ],
        out_specs=[
            pl.BlockSpec((1, nC, C), lambda r, bh: (r * BH + bh, 0, 0)),
            pl.BlockSpec((1, nC, C), lambda r, bh: (r * BH + bh, 0, 0)),
            pl.BlockSpec((1, nC, C), lambda r, bh: (r * BH + bh, 0, 0)),
        ],
        out_shape=[
            jax.ShapeDtypeStruct((RBH, nC, C), jnp.int32),
            jax.ShapeDtypeStruct((RBH, nC, C), jnp.int32),
            jax.ShapeDtypeStruct((RBH, nC, C), jnp.int32),
        ],
        compiler_params=pltpu.CompilerParams(
            dimension_semantics=("arbitrary", "arbitrary")),
    )(qbh, hash_w)


def _make_sc_scatter():
    mesh = plsc.VectorSubcoreMesh(core_axis_name="c", subcore_axis_name="s", num_cores=2, num_subcores=16)

    @functools.partial(
        pl.kernel,
        out_type=[
            jax.ShapeDtypeStruct((RBH * T, Dh), jnp.float32),
            jax.ShapeDtypeStruct((RBH * T, Dh), jnp.float32),
            jax.ShapeDtypeStruct((RBH * T,), jnp.int32),
        ],
        mesh=mesh,
        scratch_types=[
            pltpu.VMEM((T,), jnp.int32),
            pltpu.VMEM((T,), jnp.int32),
            pltpu.VMEM((T,), jnp.int32),
            pltpu.VMEM((nC, C), jnp.int32),
            pltpu.VMEM((512, Dh), jnp.float32),
            pltpu.VMEM((512, Dh), jnp.float32),
            pltpu.SemaphoreType.DMA,
            pltpu.SemaphoreType.DMA,
        ],
        compiler_params=pltpu.CompilerParams(needs_layout_passes=False, use_tc_tiling_on_sc=False),
    )
    def sc_scatter(qrows_hbm, vrows_hbm, dstg_hbm, undo_hbm, meta_hbm,
                   sq_hbm, sv_hbm, smeta_hbm,
                   undo_v, meta_v, smeta_v, idx_all, qrows_v, vrows_v,
                   sem_q, sem_v):
        wid = lax.axis_index("s") * 2 + lax.axis_index("c")
        for it in range(2):
            task = wid * 2 + it
            bh = lax.rem(task, BH)
            src_base = bh * T
            pltpu.sync_copy(dstg_hbm.at[task], idx_all)
            pltpu.sync_copy(undo_hbm.at[pl.ds(task * T, T)], undo_v)
            pltpu.sync_copy(meta_hbm.at[pl.ds(task * T, T)], meta_v)

            def meta_body(i, carry):
                u16 = undo_v[pl.ds(i * 16, 16)]
                m16 = meta_v[pl.ds(i * 16, 16)]
                plsc.store_scatter(smeta_v, [u16], m16)
                return carry

            lax.fori_loop(0, T // 16, meta_body, 0)

            def row_body(g, carry):
                pltpu.sync_copy(qrows_hbm.at[pl.ds(src_base + g * 512, 512)],
                                qrows_v)
                pltpu.sync_copy(vrows_hbm.at[pl.ds(src_base + g * 512, 512)],
                                vrows_v)
                descs = []
                for j in range(4):
                    ix = idx_all.at[g * 4 + j]
                    descs.append(pltpu.async_copy(
                        qrows_v.at[pl.ds(j * C, C)], sq_hbm.at[ix], sem_q))
                    descs.append(pltpu.async_copy(
                        vrows_v.at[pl.ds(j * C, C)], sv_hbm.at[ix], sem_v))
                for d in descs:
                    d.wait()
                return carry

            lax.fori_loop(0, 8, row_body, 0)
            pltpu.sync_copy(smeta_v, smeta_hbm.at[pl.ds(task * T, T)])

    return sc_scatter


_sc_scatter_cache = functools.cache(_make_sc_scatter)


def _attn_body(sq_ref, sv_ref, met_ref, so_ref, lse_ref):
    sq = sq_ref[0]
    sv = sv_ref[0]
    met = met_ref[0]

    def prev(x):
        return jnp.concatenate([x[nC - 1:], x[:nC - 1]], axis=0)

    nrm = jnp.sqrt(jnp.sum(sq * sq, axis=-1, keepdims=True))
    k = sq / (nrm + 1e-6)
    k2 = jnp.concatenate([k, prev(k)], axis=1)
    v2 = jnp.concatenate([sv, prev(sv)], axis=1)
    m2 = jnp.concatenate([met, prev(met)], axis=1)

    s = lax.dot_general(sq, k2, (((2,), (2,)), ((0,), (0,))),
                        preferred_element_type=jnp.float32) * SCALING
    qm = met[:, :, None]
    km = m2[:, None, :]
    s = jnp.where(qm == km, -1e8, s)
    s = jnp.where((qm >> LOG2T) != (km >> LOG2T), -1e16, s)
    mx = jnp.max(s, axis=-1, keepdims=True)
    p = jnp.exp(s - mx)
    den = jnp.sum(p, axis=-1, keepdims=True)
    o = lax.dot_general(p, v2, (((2,), (1,)), ((0,), (0,))),
                        preferred_element_type=jnp.float32) / den
    so_ref[0] = o
    lse_ref[0] = mx[..., 0] + jnp.log(den[..., 0])


def _attn(sq4, sv4, smeta3):
    return pl.pallas_call(
        _attn_body,
        grid=(RBH,),
        in_specs=[
            pl.BlockSpec((1, nC, C, Dh), lambda i: (i, 0, 0, 0)),
            pl.BlockSpec((1, nC, C, Dh), lambda i: (i, 0, 0, 0)),
            pl.BlockSpec((1, nC, C), lambda i: (i, 0, 0)),
        ],
        out_specs=[
            pl.BlockSpec((1, nC, C, Dh), lambda i: (i, 0, 0, 0)),
            pl.BlockSpec((1, nC, C), lambda i: (i, 0, 0)),
        ],
        out_shape=[
            jax.ShapeDtypeStruct((RBH, nC, C, Dh), jnp.float32),
            jax.ShapeDtypeStruct((RBH, nC, C), jnp.float32),
        ],
        compiler_params=pltpu.CompilerParams(
            dimension_semantics=("arbitrary",)),
    )(sq4, sv4, smeta3)


def _make_sc_gather():
    mesh = plsc.VectorSubcoreMesh(core_axis_name="c", subcore_axis_name="s", num_cores=2, num_subcores=16)

    @functools.partial(
        pl.kernel,
        out_type=[
            jax.ShapeDtypeStruct((RBH * T, Dh), jnp.float32),
            jax.ShapeDtypeStruct((RBH * T,), jnp.float32),
        ],
        mesh=mesh,
        scratch_types=[
            pltpu.VMEM((T,), jnp.int32),
            pltpu.VMEM((T,), jnp.float32),
            pltpu.VMEM((T,), jnp.float32),
            pltpu.VMEM((nC, C), jnp.int32),
            pltpu.VMEM((512, Dh), jnp.float32),
            pltpu.SemaphoreType.DMA,
        ],
        compiler_params=pltpu.CompilerParams(needs_layout_passes=False, use_tc_tiling_on_sc=False),
    )
    def sc_gather(so_hbm, slse_hbm, dstg_hbm, undo_hbm,
                  o_hbm, lse_hbm,
                  undo_v, slse_v, lse_v, idx_all, rows_v, sem):
        wid = lax.axis_index("s") * 2 + lax.axis_index("c")
        for it in range(2):
            task = wid * 2 + it
            pltpu.sync_copy(dstg_hbm.at[task], idx_all)
            pltpu.sync_copy(undo_hbm.at[pl.ds(task * T, T)], undo_v)
            pltpu.sync_copy(slse_hbm.at[pl.ds(task * T, T)], slse_v)

            def lse_body(i, carry):
                u16 = undo_v[pl.ds(i * 16, 16)]
                lse_v[pl.ds(i * 16, 16)] = plsc.load_gather(slse_v, [u16])
                return carry

            lax.fori_loop(0, T // 16, lse_body, 0)

            def row_body(g, carry):
                descs = []
                for j in range(4):
                    descs.append(pltpu.async_copy(
                        so_hbm.at[idx_all.at[g * 4 + j]],
                        rows_v.at[pl.ds(j * C, C)], sem))
                for d in descs:
                    d.wait()
                pltpu.sync_copy(rows_v,
                                o_hbm.at[pl.ds(task * T + g * 512, 512)])
                return carry

            lax.fori_loop(0, 8, row_body, 0)
            pltpu.sync_copy(lse_v, lse_hbm.at[pl.ds(task * T, T)])

    return sc_gather


_sc_gather_cache = functools.cache(_make_sc_gather)


TT6 = 256


def _combine_body(o_ref, lse_ref, wout_ref, bout_ref, out_ref):
    l = lse_ref[...]
    mx = jnp.max(l, axis=0, keepdims=True)
    w = jnp.exp(l - mx)
    w = w / jnp.sum(w, axis=0, keepdims=True)
    o = o_ref[...]
    ov = jnp.sum(w[..., None] * o, axis=0)
    acc = jnp.zeros((TT6, E), jnp.float32)
    for h in range(H):
        acc = acc + lax.dot_general(
            ov[h], wout_ref[h * Dh:(h + 1) * Dh, :],
            (((1,), (0,)), ((), ())), preferred_element_type=jnp.float32)
    out_ref[0] = acc + bout_ref[...]


def _combine(o6, lse6, woutT, bout2):
    return pl.pallas_call(
        _combine_body,
        grid=(T // TT6, B),
        in_specs=[
            pl.BlockSpec((R, H, TT6, Dh), lambda t, b: (0, b, t, 0)),
            pl.BlockSpec((R, H, TT6), lambda t, b: (0, b, t)),
            pl.BlockSpec((E, E), lambda t, b: (0, 0)),
            pl.BlockSpec((1, E), lambda t, b: (0, 0)),
        ],
        out_specs=pl.BlockSpec((1, TT6, E), lambda t, b: (b, t, 0)),
        out_shape=jax.ShapeDtypeStruct((B, T, E), jnp.float32),
        compiler_params=pltpu.CompilerParams(
            dimension_semantics=("arbitrary", "arbitrary")),
    )(o6, lse6, woutT, bout2)


def kernel(query, key, value, Wq, bq, Wv, bv, Wout, bout, hash_w):
    del key
    qbh, vbh = _proj(query.transpose(1, 0, 2), value.transpose(1, 0, 2),
                     Wq, bq, Wv, bv)
    undo, meta, dstg = _hashperm(qbh, hash_w)
    sq, sv, smeta = _sc_scatter_cache()(
        qbh.reshape(BH * T, Dh), vbh.reshape(BH * T, Dh),
        dstg, undo.reshape(RBH * T), meta.reshape(RBH * T))
    so, slse = _attn(sq.reshape(RBH, nC, C, Dh), sv.reshape(RBH, nC, C, Dh),
                     smeta.reshape(RBH, nC, C))
    orows, lse = _sc_gather_cache()(so.reshape(RBH * T, Dh), slse.reshape(RBH * T),
                            dstg, undo.reshape(RBH * T))
    out = _combine(orows.reshape(R, BH, T, Dh),
                   lse.reshape(R, BH, T), Wout.T, bout.reshape(1, E))
    return out.transpose(1, 0, 2)


# speedup vs baseline: 13.2810x; 1.5992x over previous
import functools

import jax
import jax.numpy as jnp
from jax import lax
from jax.experimental import pallas as pl
from jax.experimental.pallas import tpu as pltpu
from jax.experimental.pallas import tpu_sc as plsc

T, B, E, H = 4096, 2, 1024, 16
Dh = E // H
D2 = 2 * Dh
R, NH, C = 2, 16, 128
nC = T // C
HB = H * B
RBH = R * HB
SCALING = Dh ** -0.5
LOG2T = 12


TP = 1024


def _proj_body(xq_ref, xv_ref, wq_ref, wv_ref, bq_ref, bv_ref, qv_ref):
    wq = wq_ref[...]
    wv = wv_ref[...]
    dn = (((1,), (1,)), ((), ()))
    for b in range(B):
        xq = xq_ref[:, b, :]
        xv = xv_ref[:, b, :]
        q = lax.dot_general(xq, wq, dn, preferred_element_type=jnp.float32)
        v = lax.dot_general(xv, wv, dn, preferred_element_type=jnp.float32)
        q = q + bq_ref[0, 0][None, :]
        v = v + bv_ref[0, 0][None, :]
        qv_ref[0, b] = jnp.concatenate([q, v], axis=1)


def _proj(query, value, Wq, bq, Wv, bv):
    bq3 = bq.reshape(H, 1, Dh)
    bv3 = bv.reshape(H, 1, Dh)
    return pl.pallas_call(
        _proj_body,
        grid=(T // TP, H),
        in_specs=[
            pl.BlockSpec((TP, B, E), lambda t, h: (t, 0, 0)),
            pl.BlockSpec((TP, B, E), lambda t, h: (t, 0, 0)),
            pl.BlockSpec((Dh, E), lambda t, h: (h, 0)),
            pl.BlockSpec((Dh, E), lambda t, h: (h, 0)),
            pl.BlockSpec((1, 1, Dh), lambda t, h: (h, 0, 0)),
            pl.BlockSpec((1, 1, Dh), lambda t, h: (h, 0, 0)),
        ],
        out_specs=pl.BlockSpec((1, B, TP, D2), lambda t, h: (h, 0, t, 0)),
        out_shape=jax.ShapeDtypeStruct((H, B, T, D2), jnp.float32),
        compiler_params=pltpu.CompilerParams(
            dimension_semantics=("arbitrary", "arbitrary")),
    )(query, value, Wq, Wv, bq3, bv3)


def _hashperm_body(qv_ref, w_ref, mlt_ref, jlt_ref, ub_ref,
                   undo_ref, meta_ref, dstg_ref):
    r = pl.program_id(0)
    hb = pl.program_id(1)
    q2 = qv_ref[0][:, :Dh]
    w = w_ref[0, 0]
    wf = jnp.concatenate([w, -w], axis=1)
    rot = lax.dot_general(q2, wf, (((1,), (0,)), ((), ())),
                          preferred_element_type=jnp.float32)
    mx = jnp.max(rot, axis=-1, keepdims=True)
    oh0 = (rot >= mx).astype(jnp.float32)
    ub = ub_ref[...]
    pstrict = lax.dot_general(oh0, ub, (((1,), (0,)), ((), ())),
                              preferred_element_type=jnp.float32)
    ohf = oh0 * (pstrict == 0.0).astype(jnp.float32)
    oh_j = ohf.reshape(nC, C, NH)
    lane = lax.broadcasted_iota(jnp.int32, (nC, C, NH), 2)
    codes = jnp.sum(oh_j * lane.astype(jnp.float32), axis=-1).astype(jnp.int32)

    S = lax.dot_general(mlt_ref[...], ohf, (((1,), (0,)), ((), ())),
                        preferred_element_type=jnp.float32)
    totals = lax.dot_general(jnp.ones((1, T), jnp.float32), ohf,
                             (((1,), (0,)), ((), ())),
                             preferred_element_type=jnp.float32)
    offs = lax.dot_general(totals, ub, (((1,), (0,)), ((), ())),
                           preferred_element_type=jnp.float32,
                           precision=jax.lax.Precision.HIGHEST)
    same = lax.dot_general(oh_j, oh_j, (((2,), (2,)), ((0,), (0,))),
                           preferred_element_type=jnp.float32)
    wpick = jnp.sum(same * jlt_ref[...][None], axis=-1)
    base = jnp.sum(oh_j * (S + offs)[:, None, :], axis=-1)
    undo = (wpick + base + 0.5).astype(jnp.int32)

    ti = (lax.broadcasted_iota(jnp.int32, (nC, C), 0) << 7) \
        + lax.broadcasted_iota(jnp.int32, (nC, C), 1)
    meta = (codes << LOG2T) + ti
    undo_ref[0] = undo
    meta_ref[0] = meta
    dstg_ref[0] = undo + (r * HB + hb) * T


def _hashperm(qv, hash_w, mlt, jlt, ub):
    qv2 = qv.reshape(HB, T, D2)
    return pl.pallas_call(
        _hashperm_body,
        grid=(R, HB),
        in_specs=[
            pl.BlockSpec((1, T, D2), lambda r, hb: (hb, 0, 0)),
            pl.BlockSpec((1, 1, Dh, NH // 2), lambda r, hb: (r, hb // B, 0, 0)),
            pl.BlockSpec((nC, T), lambda r, hb: (0, 0)),
            pl.BlockSpec((C, C), lambda r, hb: (0, 0)),
            pl.BlockSpec((NH, NH), lambda r, hb: (0, 0)),
        ],
        out_specs=[
            pl.BlockSpec((1, nC, C), lambda r, hb: (r * HB + hb, 0, 0)),
            pl.BlockSpec((1, nC, C), lambda r, hb: (r * HB + hb, 0, 0)),
            pl.BlockSpec((1, nC, C), lambda r, hb: (r * HB + hb, 0, 0)),
        ],
        out_shape=[
            jax.ShapeDtypeStruct((RBH, nC, C), jnp.int32),
            jax.ShapeDtypeStruct((RBH, nC, C), jnp.int32),
            jax.ShapeDtypeStruct((RBH, nC, C), jnp.int32),
        ],
        compiler_params=pltpu.CompilerParams(
            dimension_semantics=("arbitrary", "arbitrary")),
    )(qv2, hash_w, mlt, jlt, ub)


def _perm_consts():
    blk_of = jnp.arange(T, dtype=jnp.int32)[None, :] // C
    mlt = (blk_of < jnp.arange(nC, dtype=jnp.int32)[:, None]).astype(jnp.float32)
    jj = jnp.arange(C, dtype=jnp.int32)
    jlt = (jj[None, :] < jj[:, None]).astype(jnp.float32)
    cc = jnp.arange(NH, dtype=jnp.int32)
    ub = (cc[:, None] < cc[None, :]).astype(jnp.float32)
    return mlt, jlt, ub


def _make_sc_scatter():
    mesh = plsc.VectorSubcoreMesh(core_axis_name="c", subcore_axis_name="s",
                                  num_cores=2, num_subcores=16)

    @functools.partial(
        pl.kernel,
        out_type=[
            jax.ShapeDtypeStruct((RBH * T, D2), jnp.float32),
            jax.ShapeDtypeStruct((RBH * T,), jnp.int32),
        ],
        mesh=mesh,
        scratch_types=[
            pltpu.VMEM((T,), jnp.int32),
            pltpu.VMEM((T,), jnp.int32),
            pltpu.VMEM((T,), jnp.int32),
            pltpu.VMEM((R, nC, C), jnp.int32),
            pltpu.VMEM((512, D2), jnp.float32),
            pltpu.SemaphoreType.DMA,
            pltpu.SemaphoreType.DMA,
        ],
        compiler_params=pltpu.CompilerParams(
            needs_layout_passes=False, use_tc_tiling_on_sc=False),
    )
    def sc_scatter(qv_hbm, dstg_hbm, undo_hbm, meta_hbm,
                   sqv_hbm, smeta_hbm,
                   undo_v, meta_v, smeta_v, idx2, qv_v, sem_in, sem_out):
        hb = lax.axis_index("s") * 2 + lax.axis_index("c")
        src_base = hb * T
        for r in range(R):
            task = r * HB + hb
            pltpu.sync_copy(dstg_hbm.at[task], idx2.at[r])
            pltpu.sync_copy(undo_hbm.at[pl.ds(task * T, T)], undo_v)
            pltpu.sync_copy(meta_hbm.at[pl.ds(task * T, T)], meta_v)

            def meta_body(i, carry):
                u16 = undo_v[pl.ds(i * 16, 16)]
                m16 = meta_v[pl.ds(i * 16, 16)]
                plsc.store_scatter(smeta_v, [u16], m16)
                return carry

            lax.fori_loop(0, T // 16, meta_body, 0)
            pltpu.sync_copy(smeta_v, smeta_hbm.at[pl.ds(task * T, T)])

        def row_body(g, carry):
            pltpu.async_copy(
                qv_hbm.at[pl.ds(src_base + g * 512, 512)], qv_v, sem_in).wait()
            descs = []
            for r in range(R):
                for j in range(4):
                    descs.append(pltpu.async_copy(
                        qv_v.at[pl.ds(j * C, C)],
                        sqv_hbm.at[idx2.at[r, g * 4 + j]], sem_out))
            for d in descs:
                d.wait()
            return carry

        lax.fori_loop(0, 8, row_body, 0)

    return sc_scatter


_sc_scatter_cache = functools.cache(_make_sc_scatter)


def _attn_body(sqv_ref, met_ref, so_ref, lse_ref):
    sq = sqv_ref[0][:, :, :Dh]
    sv = sqv_ref[0][:, :, Dh:]
    met = met_ref[0]

    def prev(x):
        return jnp.concatenate([x[nC - 1:], x[:nC - 1]], axis=0)

    nrm = jnp.sqrt(jnp.sum(sq * sq, axis=-1, keepdims=True))
    k = sq / (nrm + 1e-6)
    k2 = jnp.concatenate([k, prev(k)], axis=1)
    v2 = jnp.concatenate([sv, prev(sv)], axis=1)
    m2 = jnp.concatenate([met, prev(met)], axis=1)

    s = lax.dot_general(sq, k2, (((2,), (2,)), ((0,), (0,))),
                        preferred_element_type=jnp.float32) * SCALING
    qm = met[:, :, None]
    km = m2[:, None, :]
    s = jnp.where(qm == km, -1e8, s)
    s = jnp.where((qm >> LOG2T) != (km >> LOG2T), -1e16, s)
    mx = jnp.max(s, axis=-1, keepdims=True)
    p = jnp.exp(s - mx)
    den = jnp.sum(p, axis=-1, keepdims=True)
    o = lax.dot_general(p, v2, (((2,), (1,)), ((0,), (0,))),
                        preferred_element_type=jnp.float32) / den
    so_ref[0] = o
    lse_ref[0] = mx[..., 0] + jnp.log(den[..., 0])


def _attn(sqv4, smeta3):
    return pl.pallas_call(
        _attn_body,
        grid=(RBH,),
        in_specs=[
            pl.BlockSpec((1, nC, C, D2), lambda i: (i, 0, 0, 0)),
            pl.BlockSpec((1, nC, C), lambda i: (i, 0, 0)),
        ],
        out_specs=[
            pl.BlockSpec((1, nC, C, Dh), lambda i: (i, 0, 0, 0)),
            pl.BlockSpec((1, nC, C), lambda i: (i, 0, 0)),
        ],
        out_shape=[
            jax.ShapeDtypeStruct((RBH, nC, C, Dh), jnp.float32),
            jax.ShapeDtypeStruct((RBH, nC, C), jnp.float32),
        ],
        compiler_params=pltpu.CompilerParams(
            dimension_semantics=("arbitrary",)),
    )(sqv4, smeta3)


def _make_sc_gather():
    mesh = plsc.VectorSubcoreMesh(core_axis_name="c", subcore_axis_name="s",
                                  num_cores=2, num_subcores=16)

    @functools.partial(
        pl.kernel,
        out_type=[
            jax.ShapeDtypeStruct((RBH * T, Dh), jnp.float32),
            jax.ShapeDtypeStruct((RBH * T,), jnp.float32),
        ],
        mesh=mesh,
        scratch_types=[
            pltpu.VMEM((T,), jnp.int32),
            pltpu.VMEM((T,), jnp.float32),
            pltpu.VMEM((T,), jnp.float32),
            pltpu.VMEM((nC, C), jnp.int32),
            pltpu.VMEM((512, Dh), jnp.float32),
            pltpu.SemaphoreType.DMA,
        ],
        compiler_params=pltpu.CompilerParams(
            needs_layout_passes=False, use_tc_tiling_on_sc=False),
    )
    def sc_gather(so_hbm, slse_hbm, dstg_hbm, undo_hbm,
                  o_hbm, lse_hbm,
                  undo_v, slse_v, lse_v, idx_all, rows_v, sem):
        wid = lax.axis_index("s") * 2 + lax.axis_index("c")
        for it in range(2):
            task = wid * 2 + it
            pltpu.sync_copy(dstg_hbm.at[task], idx_all)
            pltpu.sync_copy(undo_hbm.at[pl.ds(task * T, T)], undo_v)
            pltpu.sync_copy(slse_hbm.at[pl.ds(task * T, T)], slse_v)

            def lse_body(i, carry):
                u16 = undo_v[pl.ds(i * 16, 16)]
                lse_v[pl.ds(i * 16, 16)] = plsc.load_gather(slse_v, [u16])
                return carry

            lax.fori_loop(0, T // 16, lse_body, 0)

            def row_body(g, carry):
                descs = []
                for j in range(4):
                    descs.append(pltpu.async_copy(
                        so_hbm.at[idx_all.at[g * 4 + j]],
                        rows_v.at[pl.ds(j * C, C)], sem))
                for d in descs:
                    d.wait()
                pltpu.sync_copy(rows_v,
                                o_hbm.at[pl.ds(task * T + g * 512, 512)])
                return carry

            lax.fori_loop(0, 8, row_body, 0)
            pltpu.sync_copy(lse_v, lse_hbm.at[pl.ds(task * T, T)])

    return sc_gather


_sc_gather_cache = functools.cache(_make_sc_gather)


TT6 = 256


def _combine_body(o_ref, lse_ref, wout_ref, bout_ref, out_ref):
    l = lse_ref[...]
    mxl = jnp.max(l, axis=0, keepdims=True)
    wts = jnp.exp(l - mxl)
    wts = wts / jnp.sum(wts, axis=0, keepdims=True)
    o = o_ref[...]
    ov = jnp.sum(wts[..., None] * o, axis=0)
    for b in range(B):
        acc = jnp.zeros((TT6, E), jnp.float32)
        for h in range(H):
            acc = acc + lax.dot_general(
                ov[h * B + b], wout_ref[:, h * Dh:(h + 1) * Dh],
                (((1,), (1,)), ((), ())), preferred_element_type=jnp.float32)
        out_ref[:, b, :] = acc + bout_ref[...]


def _combine(o6, lse6, wout, bout2):
    return pl.pallas_call(
        _combine_body,
        grid=(T // TT6,),
        in_specs=[
            pl.BlockSpec((R, HB, TT6, Dh), lambda t: (0, 0, t, 0)),
            pl.BlockSpec((R, HB, TT6), lambda t: (0, 0, t)),
            pl.BlockSpec((E, E), lambda t: (0, 0)),
            pl.BlockSpec((1, E), lambda t: (0, 0)),
        ],
        out_specs=pl.BlockSpec((TT6, B, E), lambda t: (t, 0, 0)),
        out_shape=jax.ShapeDtypeStruct((T, B, E), jnp.float32),
        compiler_params=pltpu.CompilerParams(
            dimension_semantics=("arbitrary",)),
    )(o6, lse6, wout, bout2)


def kernel(query, key, value, Wq, bq, Wv, bv, Wout, bout, hash_w):
    del key
    qv = _proj(query, value, Wq, bq, Wv, bv)
    mlt, jlt, ub = _perm_consts()
    undo, meta, dstg = _hashperm(qv, hash_w, mlt, jlt, ub)
    sqv, smeta = _sc_scatter_cache()(
        qv.reshape(HB * T, D2), dstg, undo.reshape(RBH * T),
        meta.reshape(RBH * T))
    so, slse = _attn(sqv.reshape(RBH, nC, C, D2), smeta.reshape(RBH, nC, C))
    orows, lse = _sc_gather_cache()(so.reshape(RBH * T, Dh),
                                    slse.reshape(RBH * T),
                                    dstg, undo.reshape(RBH * T))
    return _combine(orows.reshape(R, HB, T, Dh), lse.reshape(R, HB, T),
                    Wout, bout.reshape(1, E))
